# baseline (speedup 1.0000x reference)
import jax
import jax.numpy as jnp
from jax import lax
from jax.experimental import pallas as pl
from jax.experimental.pallas import tpu as pltpu

_HALF_ACC = 256
_HALF_IN = 41920
_BK = 512
_K_TILES = _HALF_IN // _BK
_K_MAIN = _K_TILES * _BK
_K_REM = _HALF_IN - _K_MAIN
_BM = 512

_DN = (((1,), (1,)), ((), ()))


def _nn_body(wf_ref, bf_ref, stm_ref, Ww_ref, Wb_ref,
             wft_ref, bft_ref, Wwt_ref, Wbt_ref,
             bw_ref, bb_ref, W1_ref, b1_ref, W2_ref, b2_ref, Wo_ref, bo_ref,
             out_ref, accw_ref, accb_ref):
    k = pl.program_id(1)

    pw = lax.dot_general(wf_ref[...], Ww_ref[...], _DN,
                         preferred_element_type=jnp.float32)
    pb = lax.dot_general(bf_ref[...], Wb_ref[...], _DN,
                         preferred_element_type=jnp.float32)

    @pl.when(k == 0)
    def _init():
        accw_ref[...] = pw
        accb_ref[...] = pb

    @pl.when(k > 0)
    def _accum():
        accw_ref[...] += pw
        accb_ref[...] += pb

    @pl.when(k == _K_TILES - 1)
    def _tail():
        accw = (accw_ref[...] + bw_ref[...]
                + lax.dot_general(wft_ref[...], Wwt_ref[...], _DN,
                                  preferred_element_type=jnp.float32))
        accb = (accb_ref[...] + bb_ref[...]
                + lax.dot_general(bft_ref[...], Wbt_ref[...], _DN,
                                  preferred_element_type=jnp.float32))
        stm = stm_ref[...]
        h1 = jnp.clip((1.0 - stm) * accw + stm * accb, 0.0, 1.0)
        h2 = jnp.clip(stm * accw + (1.0 - stm) * accb, 0.0, 1.0)
        W1 = W1_ref[...]
        o1 = (lax.dot_general(h1, W1[:, :_HALF_ACC], _DN,
                              preferred_element_type=jnp.float32)
              + lax.dot_general(h2, W1[:, _HALF_ACC:], _DN,
                                preferred_element_type=jnp.float32)
              + b1_ref[...])
        i2 = jnp.clip(o1, 0.0, 1.0)
        o2 = lax.dot_general(i2, W2_ref[...], _DN,
                             preferred_element_type=jnp.float32) + b2_ref[...]
        io = jnp.clip(o2, 0.0, 1.0)
        out_ref[...] = lax.dot_general(io, Wo_ref[...], _DN,
                                       preferred_element_type=jnp.float32) + bo_ref[0]


def _pad_tail(x):
    return jnp.pad(x[:, _K_MAIN:], ((0, 0), (0, _BK - _K_REM)))


def kernel(white_features, black_features, stm, Ww, bw, Wb, bb,
           W1, b1, W2, b2, Wo, bo):
    batch = white_features.shape[0]
    bm_tiles = batch // _BM
    out = pl.pallas_call(
        _nn_body,
        grid=(bm_tiles, _K_TILES),
        in_specs=[
            pl.BlockSpec((_BM, _BK), lambda p, k: (p, k)),
            pl.BlockSpec((_BM, _BK), lambda p, k: (p, k)),
            pl.BlockSpec((_BM, 1), lambda p, k: (p, 0)),
            pl.BlockSpec((_HALF_ACC, _BK), lambda p, k: (0, k)),
            pl.BlockSpec((_HALF_ACC, _BK), lambda p, k: (0, k)),
            pl.BlockSpec((_BM, _BK), lambda p, k: (p, 0)),
            pl.BlockSpec((_BM, _BK), lambda p, k: (p, 0)),
            pl.BlockSpec((_HALF_ACC, _BK), lambda p, k: (0, 0)),
            pl.BlockSpec((_HALF_ACC, _BK), lambda p, k: (0, 0)),
            pl.BlockSpec((1, _HALF_ACC), lambda p, k: (0, 0)),
            pl.BlockSpec((1, _HALF_ACC), lambda p, k: (0, 0)),
            pl.BlockSpec((32, 2 * _HALF_ACC), lambda p, k: (0, 0)),
            pl.BlockSpec((1, 32), lambda p, k: (0, 0)),
            pl.BlockSpec((32, 32), lambda p, k: (0, 0)),
            pl.BlockSpec((1, 32), lambda p, k: (0, 0)),
            pl.BlockSpec((128, 32), lambda p, k: (0, 0)),
            pl.BlockSpec(memory_space=pltpu.SMEM),
        ],
        out_specs=pl.BlockSpec((_BM, 128), lambda p, k: (p, 0)),
        out_shape=jax.ShapeDtypeStruct((batch, 128), jnp.float32),
        scratch_shapes=[
            pltpu.VMEM((_BM, _HALF_ACC), jnp.float32),
            pltpu.VMEM((_BM, _HALF_ACC), jnp.float32),
        ],
        compiler_params=pltpu.CompilerParams(
            dimension_semantics=("parallel", "arbitrary"),
        ),
    )(white_features, black_features, stm, Ww, Wb,
      _pad_tail(white_features), _pad_tail(black_features),
      _pad_tail(Ww), _pad_tail(Wb),
      bw.reshape(1, -1), bb.reshape(1, -1),
      W1, b1.reshape(1, -1), W2, b2.reshape(1, -1),
      jnp.pad(Wo, ((0, 128 - Wo.shape[0]), (0, 0))), bo)
    return out[:, :1]


# speedup vs baseline: 4.3307x; 4.3307x over previous
import jax
import jax.numpy as jnp
from jax import lax
from jax.experimental import pallas as pl
from jax.experimental.pallas import tpu as pltpu

_HALF_ACC = 256
_HALF_IN = 41920
_BK = 1048
_K_TILES = _HALF_IN // _BK

_DNT = (((0,), (0,)), ((), ()))
_DN = (((1,), (1,)), ((), ()))


def _nn_body(wf_ref, bf_ref, stm_ref, Ww_ref, Wb_ref,
             bw_ref, bb_ref, W1_ref, b1_ref, W2_ref, b2_ref, Wo_ref, bo_ref,
             out_ref, accw_ref, accb_ref):
    k = pl.program_id(0)

    pw = lax.dot_general(wf_ref[...], Ww_ref[...], _DNT,
                         preferred_element_type=jnp.float32)
    pb = lax.dot_general(bf_ref[...], Wb_ref[...], _DNT,
                         preferred_element_type=jnp.float32)

    @pl.when(k == 0)
    def _init():
        accw_ref[...] = pw
        accb_ref[...] = pb

    @pl.when(k > 0)
    def _accum():
        accw_ref[...] += pw
        accb_ref[...] += pb

    @pl.when(k == _K_TILES - 1)
    def _tail():
        accw = accw_ref[...] + bw_ref[...]
        accb = accb_ref[...] + bb_ref[...]
        stm = stm_ref[...]
        h1 = jnp.clip((1.0 - stm) * accw + stm * accb, 0.0, 1.0)
        h2 = jnp.clip(stm * accw + (1.0 - stm) * accb, 0.0, 1.0)
        W1 = W1_ref[...]
        o1 = (lax.dot_general(h1, W1[:, :_HALF_ACC], _DN,
                              preferred_element_type=jnp.float32)
              + lax.dot_general(h2, W1[:, _HALF_ACC:], _DN,
                                preferred_element_type=jnp.float32)
              + b1_ref[...])
        i2 = jnp.clip(o1, 0.0, 1.0)
        o2 = lax.dot_general(i2, W2_ref[...], _DN,
                             preferred_element_type=jnp.float32) + b2_ref[...]
        io = jnp.clip(o2, 0.0, 1.0)
        out_ref[...] = lax.dot_general(io, Wo_ref[...], _DN,
                                       preferred_element_type=jnp.float32) + bo_ref[0]


def kernel(white_features, black_features, stm, Ww, bw, Wb, bb,
           W1, b1, W2, b2, Wo, bo):
    batch = white_features.shape[0]
    out = pl.pallas_call(
        _nn_body,
        grid=(_K_TILES,),
        in_specs=[
            pl.BlockSpec((_BK, batch), lambda k: (k, 0)),
            pl.BlockSpec((_BK, batch), lambda k: (k, 0)),
            pl.BlockSpec((batch, 1), lambda k: (0, 0)),
            pl.BlockSpec((_BK, _HALF_ACC), lambda k: (k, 0)),
            pl.BlockSpec((_BK, _HALF_ACC), lambda k: (k, 0)),
            pl.BlockSpec((1, _HALF_ACC), lambda k: (0, 0)),
            pl.BlockSpec((1, _HALF_ACC), lambda k: (0, 0)),
            pl.BlockSpec((32, 2 * _HALF_ACC), lambda k: (0, 0)),
            pl.BlockSpec((1, 32), lambda k: (0, 0)),
            pl.BlockSpec((32, 32), lambda k: (0, 0)),
            pl.BlockSpec((1, 32), lambda k: (0, 0)),
            pl.BlockSpec((128, 32), lambda k: (0, 0)),
            pl.BlockSpec(memory_space=pltpu.SMEM),
        ],
        out_specs=pl.BlockSpec((batch, 128), lambda k: (0, 0)),
        out_shape=jax.ShapeDtypeStruct((batch, 128), jnp.float32),
        scratch_shapes=[
            pltpu.VMEM((batch, _HALF_ACC), jnp.float32),
            pltpu.VMEM((batch, _HALF_ACC), jnp.float32),
        ],
        compiler_params=pltpu.CompilerParams(
            dimension_semantics=("arbitrary",),
        ),
    )(white_features.T, black_features.T, stm, Ww.T, Wb.T,
      bw.reshape(1, -1), bb.reshape(1, -1),
      W1, b1.reshape(1, -1), W2, b2.reshape(1, -1),
      jnp.pad(Wo, ((0, 128 - Wo.shape[0]), (0, 0))), bo)
    return out[:, :1]


# speedup vs baseline: 4.4343x; 1.0239x over previous
import jax
import jax.numpy as jnp
from jax import lax
from jax.experimental import pallas as pl
from jax.experimental.pallas import tpu as pltpu

_HALF_ACC = 256
_HALF_IN = 41920
_BK = 2096
_K_TILES = _HALF_IN // _BK

_DNT = (((0,), (0,)), ((), ()))
_DN = (((1,), (1,)), ((), ()))


def _nn_body(wf_ref, bf_ref, stm_ref, Ww_ref, Wb_ref,
             bw_ref, bb_ref, W1_ref, b1_ref, W2_ref, b2_ref, Wo_ref, bo_ref,
             out_ref, accw_ref, accb_ref):
    k = pl.program_id(0)

    pw = lax.dot_general(wf_ref[...], Ww_ref[...], _DNT,
                         preferred_element_type=jnp.float32)
    pb = lax.dot_general(bf_ref[...], Wb_ref[...], _DNT,
                         preferred_element_type=jnp.float32)

    @pl.when(k == 0)
    def _init():
        accw_ref[...] = pw
        accb_ref[...] = pb

    @pl.when(k > 0)
    def _accum():
        accw_ref[...] += pw
        accb_ref[...] += pb

    @pl.when(k == _K_TILES - 1)
    def _tail():
        accw = accw_ref[...] + bw_ref[...]
        accb = accb_ref[...] + bb_ref[...]
        stm = stm_ref[...]
        h1 = jnp.clip((1.0 - stm) * accw + stm * accb, 0.0, 1.0)
        h2 = jnp.clip(stm * accw + (1.0 - stm) * accb, 0.0, 1.0)
        W1 = W1_ref[...]
        o1 = (lax.dot_general(h1, W1[:, :_HALF_ACC], _DN,
                              preferred_element_type=jnp.float32)
              + lax.dot_general(h2, W1[:, _HALF_ACC:], _DN,
                                preferred_element_type=jnp.float32)
              + b1_ref[...])
        i2 = jnp.clip(o1, 0.0, 1.0)
        o2 = lax.dot_general(i2, W2_ref[...], _DN,
                             preferred_element_type=jnp.float32) + b2_ref[...]
        io = jnp.clip(o2, 0.0, 1.0)
        out_ref[...] = lax.dot_general(io, Wo_ref[...], _DN,
                                       preferred_element_type=jnp.float32) + bo_ref[0]


def kernel(white_features, black_features, stm, Ww, bw, Wb, bb,
           W1, b1, W2, b2, Wo, bo):
    batch = white_features.shape[0]
    out = pl.pallas_call(
        _nn_body,
        grid=(_K_TILES,),
        in_specs=[
            pl.BlockSpec((_BK, batch), lambda k: (k, 0)),
            pl.BlockSpec((_BK, batch), lambda k: (k, 0)),
            pl.BlockSpec((batch, 1), lambda k: (0, 0)),
            pl.BlockSpec((_BK, _HALF_ACC), lambda k: (k, 0)),
            pl.BlockSpec((_BK, _HALF_ACC), lambda k: (k, 0)),
            pl.BlockSpec((1, _HALF_ACC), lambda k: (0, 0)),
            pl.BlockSpec((1, _HALF_ACC), lambda k: (0, 0)),
            pl.BlockSpec((32, 2 * _HALF_ACC), lambda k: (0, 0)),
            pl.BlockSpec((1, 32), lambda k: (0, 0)),
            pl.BlockSpec((32, 32), lambda k: (0, 0)),
            pl.BlockSpec((1, 32), lambda k: (0, 0)),
            pl.BlockSpec((128, 32), lambda k: (0, 0)),
            pl.BlockSpec(memory_space=pltpu.SMEM),
        ],
        out_specs=pl.BlockSpec((batch, 128), lambda k: (0, 0)),
        out_shape=jax.ShapeDtypeStruct((batch, 128), jnp.float32),
        scratch_shapes=[
            pltpu.VMEM((batch, _HALF_ACC), jnp.float32),
            pltpu.VMEM((batch, _HALF_ACC), jnp.float32),
        ],
        compiler_params=pltpu.CompilerParams(
            dimension_semantics=("arbitrary",),
        ),
    )(white_features.T, black_features.T, stm, Ww.T, Wb.T,
      bw.reshape(1, -1), bb.reshape(1, -1),
      W1, b1.reshape(1, -1), W2, b2.reshape(1, -1),
      jnp.pad(Wo, ((0, 128 - Wo.shape[0]), (0, 0))), bo)
    return out[:, :1]
